# Initial kernel scaffold; baseline (speedup 1.0000x reference)
import jax
import jax.numpy as jnp
from jax.experimental import pallas as pl


def kernel(weight, edge_index, edge_weight):
    raise NotImplementedError("write your pallas kernel here")


# speedup vs baseline: 1.0004x; 1.0004x over previous
import functools

import jax
import jax.numpy as jnp
from jax.experimental import pallas as pl

N_NODES = 100000
EMB_DIM = 50
N_EDGES = 1600000
EPS = 1e-7
MIN_NORM = 1e-15
ROWS = 10000


def _tangent_body(w_ref, o_ref):
    w = w_ref[...]
    y = w[:, 1:]
    y_sq = jnp.sum(y * y, axis=1, keepdims=True)
    x0 = jnp.sqrt(jnp.clip(1.0 + y_sq, EPS, None))
    y_norm = jnp.clip(jnp.sqrt(y_sq), MIN_NORM, None)
    theta = jnp.clip(x0, 1.0 + EPS, None)
    acosh = jnp.log(theta + jnp.sqrt(theta * theta - 1.0))
    rest = acosh * y / y_norm
    o_ref[...] = jnp.concatenate([jnp.zeros_like(x0), rest], axis=1)


def _decode_body(h_ref, o_ref):
    h = h_ref[...]
    x = h[:, 1:]
    x_sq = jnp.sum(x * x, axis=1, keepdims=True)
    x_norm = jnp.clip(jnp.sqrt(x_sq), MIN_NORM, None)
    e = jnp.exp(x_norm)
    ei = 1.0 / e
    sinh = 0.5 * (e - ei)
    rest = sinh * x / x_norm
    r_sq = jnp.sum(rest * rest, axis=1, keepdims=True)
    x0 = jnp.sqrt(jnp.clip(1.0 + r_sq, EPS, None))
    o_ref[...] = jnp.concatenate([x0, rest], axis=1)


def _rowwise(body, x):
    n = x.shape[0]
    return pl.pallas_call(
        body,
        grid=(n // ROWS,),
        in_specs=[pl.BlockSpec((ROWS, x.shape[1]), lambda i: (i, 0))],
        out_specs=pl.BlockSpec((ROWS, x.shape[1]), lambda i: (i, 0)),
        out_shape=jax.ShapeDtypeStruct(x.shape, x.dtype),
    )(x)


def _spmm(src, dst, w, x):
    msgs = w[:, None] * jnp.take(x, src, axis=0)
    return jax.ops.segment_sum(msgs, dst, num_segments=N_NODES)


@jax.jit
def kernel(weight, edge_index, edge_weight):
    src = edge_index[0]
    dst = edge_index[1]
    t = _rowwise(_tangent_body, weight)
    o1 = _spmm(src, dst, edge_weight, t)
    o2 = _spmm(src, dst, edge_weight, o1)
    o3 = _spmm(src, dst, edge_weight, o2)
    h_t = o1 + o2 + o3
    return _rowwise(_decode_body, h_t)


# speedup vs baseline: 4.5123x; 4.5104x over previous
import functools

import jax
import jax.numpy as jnp
from jax import lax
from jax.experimental import pallas as pl
from jax.experimental.pallas import tpu as pltpu
from jax.experimental.pallas import tpu_sc as plsc

N_NODES = 100000
EMB_DIM = 50
N_EDGES = 1600000
EPS = 1e-7
MIN_NORM = 1e-15
ROWS = 10000

NSC = 2
NSUB = 16
NCHUNK = 4
TILE = 1024
MB = 128
NMB = TILE // MB
E_PAD = 1638400
EDGES_PER_SUB = E_PAD // NSUB
TILES_PER_SUB = EDGES_PER_SUB // TILE
N_PAD = 100096
ACC_ROWS = N_PAD // NSUB


def _spmm_body(xf, srcv, dst2, wv, zhbm, out,
               sbuf, dbuf, wbuf, rows, acc, gsem):
    cidx = lax.axis_index("c")
    sid = lax.axis_index("s")
    base = sid * ACC_ROWS
    e0 = sid * EDGES_PER_SUB
    d0 = sid * (EDGES_PER_SUB // MB)
    for step in range(NCHUNK // NSC):
        c = step * NSC + cidx
        cn = c * N_PAD
        pltpu.sync_copy(zhbm, acc.at[pl.ds(base, ACC_ROWS)])
        plsc.subcore_barrier()

        def tile_body(t, carry):
            toff = e0 + t * TILE
            pltpu.sync_copy(srcv.at[pl.ds(toff, TILE)], sbuf)
            pltpu.sync_copy(dst2.at[pl.ds(d0 + t * NMB, NMB)], dbuf)
            pltpu.sync_copy(wv.at[pl.ds(toff, TILE)], wbuf)

            def off(k, carry2):
                sbuf[pl.ds(k * 16, 16)] = sbuf[pl.ds(k * 16, 16)] + cn
                return carry2
            lax.fori_loop(0, TILE // 16, off, 0, unroll=8)

            copies = [
                pltpu.async_copy(
                    xf.at[sbuf.at[pl.ds(j * MB, MB)]],
                    rows.at[pl.ds(j * MB, MB)], gsem)
                for j in range(NMB)
            ]
            for cp in copies:
                cp.wait()

            def mul(g, carry2):
                wv16 = wbuf[pl.ds(g * 16, 16)]
                for l in range(16):
                    e = g * 16 + l
                    rows[e, :] = rows[e, :] * wv16[l]
                return carry2
            lax.fori_loop(0, TILE // 16, mul, 0)

            for j in range(NMB):
                pltpu.sync_copy(rows.at[pl.ds(j * MB, MB)],
                                acc.at[dbuf.at[j]], add=True)
            return carry
        lax.fori_loop(0, TILES_PER_SUB, tile_body, 0)
        plsc.subcore_barrier()
        pltpu.sync_copy(acc.at[pl.ds(base, ACC_ROWS)],
                        out.at[pl.ds(cn + base, ACC_ROWS)])
        plsc.subcore_barrier()


_spmm_sc = pl.kernel(
    _spmm_body,
    out_type=jax.ShapeDtypeStruct((NCHUNK * N_PAD, 16), jnp.float32),
    mesh=plsc.VectorSubcoreMesh(core_axis_name="c", subcore_axis_name="s"),
    compiler_params=pltpu.CompilerParams(use_tc_tiling_on_sc=False),
    scratch_types=[
        pltpu.VMEM((TILE,), jnp.int32),
        pltpu.VMEM((NMB, MB), jnp.int32),
        pltpu.VMEM((TILE,), jnp.float32),
        pltpu.VMEM((TILE, 16), jnp.float32),
        pltpu.VMEM_SHARED((N_PAD, 16), jnp.float32),
        pltpu.SemaphoreType.DMA,
    ],
)


def _tangent_body(w_ref, o_ref):
    w = w_ref[...]
    y = w[:, 1:]
    y_sq = jnp.sum(y * y, axis=1, keepdims=True)
    x0 = jnp.sqrt(jnp.clip(1.0 + y_sq, EPS, None))
    y_norm = jnp.clip(jnp.sqrt(y_sq), MIN_NORM, None)
    theta = jnp.clip(x0, 1.0 + EPS, None)
    acosh = jnp.log(theta + jnp.sqrt(theta * theta - 1.0))
    rest = acosh * y / y_norm
    o_ref[...] = jnp.concatenate([jnp.zeros_like(x0), rest], axis=1)


def _decode_body(h1_ref, h2_ref, h3_ref, o_ref):
    h = h1_ref[...] + h2_ref[...] + h3_ref[...]
    x = h[:, 1:]
    x_sq = jnp.sum(x * x, axis=1, keepdims=True)
    x_norm = jnp.clip(jnp.sqrt(x_sq), MIN_NORM, None)
    e = jnp.exp(x_norm)
    ei = 1.0 / e
    sinh = 0.5 * (e - ei)
    rest = sinh * x / x_norm
    r_sq = jnp.sum(rest * rest, axis=1, keepdims=True)
    x0 = jnp.sqrt(jnp.clip(1.0 + r_sq, EPS, None))
    o_ref[...] = jnp.concatenate([x0, rest], axis=1)


def _rowwise(body, *xs):
    n, d = xs[0].shape
    return pl.pallas_call(
        body,
        grid=(n // ROWS,),
        in_specs=[pl.BlockSpec((ROWS, d), lambda i: (i, 0)) for _ in xs],
        out_specs=pl.BlockSpec((ROWS, d), lambda i: (i, 0)),
        out_shape=jax.ShapeDtypeStruct((n, d), xs[0].dtype),
    )(*xs)


def _chunked(x):
    xp = jnp.pad(x, ((0, N_PAD - N_NODES), (0, NCHUNK * 16 - EMB_DIM)))
    return jnp.transpose(xp.reshape(N_PAD, NCHUNK, 16), (1, 0, 2)).reshape(
        NCHUNK * N_PAD, 16)


def _unchunked(xf):
    x = jnp.transpose(xf.reshape(NCHUNK, N_PAD, 16)[:, :N_NODES], (1, 0, 2))
    return x.reshape(N_NODES, NCHUNK * 16)[:, :EMB_DIM]


@jax.jit
def kernel(weight, edge_index, edge_weight):
    pad = E_PAD - N_EDGES
    src = jnp.pad(edge_index[0].astype(jnp.int32), (0, pad))
    dst2 = jnp.pad(edge_index[1].astype(jnp.int32), (0, pad)).reshape(
        E_PAD // MB, MB)
    w = jnp.pad(edge_weight, (0, pad))
    zhbm = jnp.zeros((ACC_ROWS, 16), jnp.float32)

    t = _rowwise(_tangent_body, weight)
    o1 = _spmm_sc(_chunked(t), src, dst2, w, zhbm)
    o2 = _spmm_sc(o1, src, dst2, w, zhbm)
    o3 = _spmm_sc(o2, src, dst2, w, zhbm)
    return _rowwise(_decode_body, _unchunked(o1), _unchunked(o2),
                    _unchunked(o3))


# speedup vs baseline: 5.6653x; 1.2555x over previous
import functools

import jax
import jax.numpy as jnp
from jax import lax
from jax.experimental import pallas as pl
from jax.experimental.pallas import tpu as pltpu
from jax.experimental.pallas import tpu_sc as plsc

N_NODES = 100000
EMB_DIM = 50
N_EDGES = 1600000
EPS = 1e-7
MIN_NORM = 1e-15
ROWS = 10000

NSC = 2
NSUB = 16
NCHUNK = 4
TILE = 512
MB = 128
NMB = TILE // MB
E_PAD = 1638400
EDGES_PER_SUB = E_PAD // NSUB
TILES_PER_SUB = EDGES_PER_SUB // TILE
N_PAD = 100096
ACC_ROWS = N_PAD // NSUB


def _spmm_body(xf, srcv, dst2, wv, zhbm, out,
               sbuf0, sbuf1, dbuf0, dbuf1, wbuf0, wbuf1, rows0, rows1,
               acc, isem0, isem1, gsem0, gsem1, ssem0, ssem1):
    cidx = lax.axis_index("c")
    sid = lax.axis_index("s")
    base = sid * ACC_ROWS
    e0 = sid * EDGES_PER_SUB
    d0 = sid * (EDGES_PER_SUB // MB)
    sbufs = (sbuf0, sbuf1)
    dbufs = (dbuf0, dbuf1)
    wbufs = (wbuf0, wbuf1)
    rowss = (rows0, rows1)
    isems = (isem0, isem1)
    gsems = (gsem0, gsem1)
    ssems = (ssem0, ssem1)
    nt = TILES_PER_SUB

    def in_descs(t, b):
        toff = e0 + t * TILE
        return (
            pltpu.make_async_copy(srcv.at[pl.ds(toff, TILE)], sbufs[b],
                                  isems[b]),
            pltpu.make_async_copy(dst2.at[pl.ds(d0 + t * NMB, NMB)], dbufs[b],
                                  isems[b]),
            pltpu.make_async_copy(wv.at[pl.ds(toff, TILE)], wbufs[b],
                                  isems[b]),
        )

    def scat_descs(b):
        return [
            pltpu.make_async_copy(rowss[b].at[pl.ds(j * MB, MB)],
                                  acc.at[dbufs[b].at[j]], ssems[b])
            for j in range(NMB)
        ]

    for step in range(NCHUNK // NSC):
        c = step * NSC + cidx
        cn = c * N_PAD
        pltpu.sync_copy(zhbm, acc.at[pl.ds(base, ACC_ROWS)])
        plsc.subcore_barrier()

        for d in in_descs(0, 0):
            d.start()

        def pair_body(k, carry):
            for b in (0, 1):
                t = 2 * k + b
                for d in in_descs(t, b):
                    d.wait()
                sb = sbufs[b]

                def off(q, carry2):
                    sb[pl.ds(q * 16, 16)] = sb[pl.ds(q * 16, 16)] + cn
                    return carry2
                lax.fori_loop(0, TILE // 16, off, 0, unroll=4)

                gathers = [
                    pltpu.async_copy(
                        xf.at[sbufs[b].at[pl.ds(j * MB, MB)]],
                        rowss[b].at[pl.ds(j * MB, MB)], gsems[b])
                    for j in range(NMB)
                ]

                @pl.when(t >= 1)
                def _():
                    for d in scat_descs(1 - b):
                        d.wait()

                @pl.when(t < nt - 1)
                def _():
                    for d in in_descs(t + 1, 1 - b):
                        d.start()

                for cp in gathers:
                    cp.wait()

                rw = rowss[b]
                wb = wbufs[b]

                def mul(g, carry2):
                    wv16 = wb[pl.ds(g * 16, 16)]
                    for l in range(16):
                        e = g * 16 + l
                        rw[e, :] = rw[e, :] * wv16[l]
                    return carry2
                lax.fori_loop(0, TILE // 16, mul, 0)

                for d in scat_descs(b):
                    d.start(add=True)
            return carry
        lax.fori_loop(0, nt // 2, pair_body, 0)
        for d in scat_descs((nt - 1) % 2):
            d.wait()
        plsc.subcore_barrier()
        pltpu.sync_copy(acc.at[pl.ds(base, ACC_ROWS)],
                        out.at[pl.ds(cn + base, ACC_ROWS)])
        plsc.subcore_barrier()


_spmm_sc = pl.kernel(
    _spmm_body,
    out_type=jax.ShapeDtypeStruct((NCHUNK * N_PAD, 16), jnp.float32),
    mesh=plsc.VectorSubcoreMesh(core_axis_name="c", subcore_axis_name="s"),
    compiler_params=pltpu.CompilerParams(use_tc_tiling_on_sc=False),
    scratch_types=[
        pltpu.VMEM((TILE,), jnp.int32),
        pltpu.VMEM((TILE,), jnp.int32),
        pltpu.VMEM((NMB, MB), jnp.int32),
        pltpu.VMEM((NMB, MB), jnp.int32),
        pltpu.VMEM((TILE,), jnp.float32),
        pltpu.VMEM((TILE,), jnp.float32),
        pltpu.VMEM((TILE, 16), jnp.float32),
        pltpu.VMEM((TILE, 16), jnp.float32),
        pltpu.VMEM_SHARED((N_PAD, 16), jnp.float32),
        pltpu.SemaphoreType.DMA,
        pltpu.SemaphoreType.DMA,
        pltpu.SemaphoreType.DMA,
        pltpu.SemaphoreType.DMA,
        pltpu.SemaphoreType.DMA,
        pltpu.SemaphoreType.DMA,
    ],
)


def _tangent_body(w_ref, o_ref):
    w = w_ref[...]
    y = w[:, 1:]
    y_sq = jnp.sum(y * y, axis=1, keepdims=True)
    x0 = jnp.sqrt(jnp.clip(1.0 + y_sq, EPS, None))
    y_norm = jnp.clip(jnp.sqrt(y_sq), MIN_NORM, None)
    theta = jnp.clip(x0, 1.0 + EPS, None)
    acosh = jnp.log(theta + jnp.sqrt(theta * theta - 1.0))
    rest = acosh * y / y_norm
    o_ref[...] = jnp.concatenate([jnp.zeros_like(x0), rest], axis=1)


def _decode_body(h1_ref, h2_ref, h3_ref, o_ref):
    h = h1_ref[...] + h2_ref[...] + h3_ref[...]
    x = h[:, 1:]
    x_sq = jnp.sum(x * x, axis=1, keepdims=True)
    x_norm = jnp.clip(jnp.sqrt(x_sq), MIN_NORM, None)
    e = jnp.exp(x_norm)
    ei = 1.0 / e
    sinh = 0.5 * (e - ei)
    rest = sinh * x / x_norm
    r_sq = jnp.sum(rest * rest, axis=1, keepdims=True)
    x0 = jnp.sqrt(jnp.clip(1.0 + r_sq, EPS, None))
    o_ref[...] = jnp.concatenate([x0, rest], axis=1)


def _rowwise(body, *xs):
    n, d = xs[0].shape
    return pl.pallas_call(
        body,
        grid=(n // ROWS,),
        in_specs=[pl.BlockSpec((ROWS, d), lambda i: (i, 0)) for _ in xs],
        out_specs=pl.BlockSpec((ROWS, d), lambda i: (i, 0)),
        out_shape=jax.ShapeDtypeStruct((n, d), xs[0].dtype),
    )(*xs)


def _chunked(x):
    xp = jnp.pad(x, ((0, N_PAD - N_NODES), (0, NCHUNK * 16 - EMB_DIM)))
    return jnp.transpose(xp.reshape(N_PAD, NCHUNK, 16), (1, 0, 2)).reshape(
        NCHUNK * N_PAD, 16)


def _unchunked(xf):
    x = jnp.transpose(xf.reshape(NCHUNK, N_PAD, 16)[:, :N_NODES], (1, 0, 2))
    return x.reshape(N_NODES, NCHUNK * 16)[:, :EMB_DIM]


@jax.jit
def kernel(weight, edge_index, edge_weight):
    pad = E_PAD - N_EDGES
    src = jnp.pad(edge_index[0].astype(jnp.int32), (0, pad))
    dst2 = jnp.pad(edge_index[1].astype(jnp.int32), (0, pad)).reshape(
        E_PAD // MB, MB)
    w = jnp.pad(edge_weight, (0, pad))
    zhbm = jnp.zeros((ACC_ROWS, 16), jnp.float32)

    t = _rowwise(_tangent_body, weight)
    o1 = _spmm_sc(_chunked(t), src, dst2, w, zhbm)
    o2 = _spmm_sc(o1, src, dst2, w, zhbm)
    o3 = _spmm_sc(o2, src, dst2, w, zhbm)
    return _rowwise(_decode_body, _unchunked(o1), _unchunked(o2),
                    _unchunked(o3))


# speedup vs baseline: 5.8839x; 1.0386x over previous
import jax
import jax.numpy as jnp
from jax import lax
from jax.experimental import pallas as pl
from jax.experimental.pallas import tpu as pltpu
from jax.experimental.pallas import tpu_sc as plsc

N_NODES = 100000
EMB_DIM = 50
N_EDGES = 1600000
EPS = 1e-7
MIN_NORM = 1e-15

NSC = 2
NSUB = 16
NCHUNK = 4
NLAYER = 3
TILE = 512
MB = 128
NMB = TILE // MB
E_PAD = 1638400
EDGES_PER_SUB = E_PAD // NSUB
TILES_PER_SUB = EDGES_PER_SUB // TILE
N_PAD = 100096
ACC_ROWS = N_PAD // NSUB
TROWS = 3128


def _spmm3_body(xf, srcs4, dst2, wv, zhbm, o1, o2, o3,
                sbuf0, sbuf1, dbuf0, dbuf1, wbuf0, wbuf1, rows0, rows1,
                acc, isem0, isem1, gsem0, gsem1, ssem0, ssem1):
    cidx = lax.axis_index("c")
    sid = lax.axis_index("s")
    base = sid * ACC_ROWS
    e0 = sid * EDGES_PER_SUB
    d0 = sid * (EDGES_PER_SUB // MB)
    sbufs = (sbuf0, sbuf1)
    dbufs = (dbuf0, dbuf1)
    wbufs = (wbuf0, wbuf1)
    rowss = (rows0, rows1)
    isems = (isem0, isem1)
    gsems = (gsem0, gsem1)
    ssems = (ssem0, ssem1)
    nt = TILES_PER_SUB

    def in_descs(c, t, b):
        toff = e0 + t * TILE
        return (
            pltpu.make_async_copy(srcs4.at[c].at[pl.ds(toff, TILE)],
                                  sbufs[b], isems[b]),
            pltpu.make_async_copy(dst2.at[pl.ds(d0 + t * NMB, NMB)], dbufs[b],
                                  isems[b]),
            pltpu.make_async_copy(wv.at[pl.ds(toff, TILE)], wbufs[b],
                                  isems[b]),
        )

    def scat_descs(b):
        return [
            pltpu.make_async_copy(rowss[b].at[pl.ds(j * MB, MB)],
                                  acc.at[dbufs[b].at[j]], ssems[b])
            for j in range(NMB)
        ]

    for layer in range(NLAYER):
        srcx = (xf, o1, o2)[layer]
        outx = (o1, o2, o3)[layer]
        for step in range(NCHUNK // NSC):
            c = step * NSC + cidx
            cn = c * N_PAD
            pltpu.sync_copy(zhbm, acc.at[pl.ds(base, ACC_ROWS)])
            plsc.subcore_barrier()

            for d in in_descs(c, 0, 0):
                d.start()

            def pair_body(k, carry):
                for b in (0, 1):
                    t = 2 * k + b
                    for d in in_descs(c, t, b):
                        d.wait()

                    gathers = [
                        pltpu.async_copy(
                            srcx.at[sbufs[b].at[pl.ds(j * MB, MB)]],
                            rowss[b].at[pl.ds(j * MB, MB)], gsems[b])
                        for j in range(NMB)
                    ]

                    @pl.when(t >= 1)
                    def _():
                        for d in scat_descs(1 - b):
                            d.wait()

                    @pl.when(t < nt - 1)
                    def _():
                        for d in in_descs(c, t + 1, 1 - b):
                            d.start()

                    for cp in gathers:
                        cp.wait()

                    rw = rowss[b]
                    wb = wbufs[b]

                    def mul(g, carry2):
                        wv16 = wb[pl.ds(g * 16, 16)]
                        for l in range(16):
                            e = g * 16 + l
                            rw[e, :] = rw[e, :] * wv16[l]
                        return carry2
                    lax.fori_loop(0, TILE // 16, mul, 0)

                    for d in scat_descs(b):
                        d.start(add=True)
                return carry
            lax.fori_loop(0, nt // 2, pair_body, 0)
            for d in scat_descs((nt - 1) % 2):
                d.wait()
            plsc.subcore_barrier()
            pltpu.sync_copy(acc.at[pl.ds(base, ACC_ROWS)],
                            outx.at[pl.ds(cn + base, ACC_ROWS)])
            plsc.subcore_barrier()


_ods = jax.ShapeDtypeStruct((NCHUNK * N_PAD, 16), jnp.float32)
_spmm3_sc = pl.kernel(
    _spmm3_body,
    out_type=(_ods, _ods, _ods),
    mesh=plsc.VectorSubcoreMesh(core_axis_name="c", subcore_axis_name="s"),
    compiler_params=pltpu.CompilerParams(use_tc_tiling_on_sc=False),
    scratch_types=[
        pltpu.VMEM((TILE,), jnp.int32),
        pltpu.VMEM((TILE,), jnp.int32),
        pltpu.VMEM((NMB, MB), jnp.int32),
        pltpu.VMEM((NMB, MB), jnp.int32),
        pltpu.VMEM((TILE,), jnp.float32),
        pltpu.VMEM((TILE,), jnp.float32),
        pltpu.VMEM((TILE, 16), jnp.float32),
        pltpu.VMEM((TILE, 16), jnp.float32),
        pltpu.VMEM_SHARED((N_PAD, 16), jnp.float32),
        pltpu.SemaphoreType.DMA,
        pltpu.SemaphoreType.DMA,
        pltpu.SemaphoreType.DMA,
        pltpu.SemaphoreType.DMA,
        pltpu.SemaphoreType.DMA,
        pltpu.SemaphoreType.DMA,
    ],
)


def _tangent_body(w_ref, o_ref):
    w = w_ref[...]
    y = w[:, 1:]
    y_sq = jnp.sum(y * y, axis=1, keepdims=True)
    x0 = jnp.sqrt(jnp.clip(1.0 + y_sq, EPS, None))
    y_norm = jnp.clip(jnp.sqrt(y_sq), MIN_NORM, None)
    theta = jnp.clip(x0, 1.0 + EPS, None)
    acosh = jnp.log(theta + jnp.sqrt(theta * theta - 1.0))
    rest = acosh * y / y_norm
    x64 = jnp.concatenate(
        [jnp.zeros_like(x0), rest,
         jnp.zeros((rest.shape[0], NCHUNK * 16 - EMB_DIM), rest.dtype)],
        axis=1)
    for cc in range(NCHUNK):
        o_ref[cc, :, :] = x64[:, cc * 16:(cc + 1) * 16]


def _decode_body(h1_ref, h2_ref, h3_ref, o_ref):
    h64 = jnp.concatenate(
        [h1_ref[cc, :, :] + h2_ref[cc, :, :] + h3_ref[cc, :, :]
         for cc in range(NCHUNK)], axis=1)
    x = h64[:, 1:EMB_DIM]
    x_sq = jnp.sum(x * x, axis=1, keepdims=True)
    x_norm = jnp.clip(jnp.sqrt(x_sq), MIN_NORM, None)
    e = jnp.exp(x_norm)
    ei = 1.0 / e
    sinh = 0.5 * (e - ei)
    rest = sinh * x / x_norm
    r_sq = jnp.sum(rest * rest, axis=1, keepdims=True)
    x0 = jnp.sqrt(jnp.clip(1.0 + r_sq, EPS, None))
    o_ref[...] = jnp.concatenate([x0, rest], axis=1)


_tangent_tc = pl.pallas_call(
    _tangent_body,
    grid=(N_PAD // TROWS,),
    in_specs=[pl.BlockSpec((TROWS, EMB_DIM), lambda i: (i, 0))],
    out_specs=pl.BlockSpec((NCHUNK, TROWS, 16), lambda i: (0, i, 0)),
    out_shape=jax.ShapeDtypeStruct((NCHUNK, N_PAD, 16), jnp.float32),
)

_decode_tc = pl.pallas_call(
    _decode_body,
    grid=(N_PAD // TROWS,),
    in_specs=[pl.BlockSpec((NCHUNK, TROWS, 16), lambda i: (0, i, 0))
              for _ in range(NLAYER)],
    out_specs=pl.BlockSpec((TROWS, EMB_DIM), lambda i: (i, 0)),
    out_shape=jax.ShapeDtypeStruct((N_NODES, EMB_DIM), jnp.float32),
)


@jax.jit
def kernel(weight, edge_index, edge_weight):
    pad = E_PAD - N_EDGES
    src = jnp.pad(edge_index[0].astype(jnp.int32), (0, pad))
    srcs4 = src[None, :] + (jnp.arange(NCHUNK, dtype=jnp.int32)
                            * N_PAD)[:, None]
    dst2 = jnp.pad(edge_index[1].astype(jnp.int32), (0, pad)).reshape(
        E_PAD // MB, MB)
    w = jnp.pad(edge_weight, (0, pad))
    zhbm = jnp.zeros((ACC_ROWS, 16), jnp.float32)

    xf = _tangent_tc(weight).reshape(NCHUNK * N_PAD, 16)
    o1, o2, o3 = _spmm3_sc(xf, srcs4, dst2, w, zhbm)
    return _decode_tc(o1.reshape(NCHUNK, N_PAD, 16),
                      o2.reshape(NCHUNK, N_PAD, 16),
                      o3.reshape(NCHUNK, N_PAD, 16))


# speedup vs baseline: 5.8907x; 1.0012x over previous
import jax
import jax.numpy as jnp
from jax import lax
from jax.experimental import pallas as pl
from jax.experimental.pallas import tpu as pltpu
from jax.experimental.pallas import tpu_sc as plsc

N_NODES = 100000
EMB_DIM = 50
N_EDGES = 1600000
EPS = 1e-7
MIN_NORM = 1e-15

NSC = 2
NSUB = 16
NCHUNK = 4
NLAYER = 3
TILE = 512
MB = 256
NMB = TILE // MB
E_PAD = 1638400
EDGES_PER_SUB = E_PAD // NSUB
TILES_PER_SUB = EDGES_PER_SUB // TILE
N_PAD = 100096
ACC_ROWS = N_PAD // NSUB
TROWS = 3128


def _spmm3_body(xf, srcs4, dst2, wv, zhbm, o1, o2, o3,
                sbuf0, sbuf1, dbuf0, dbuf1, wbuf0, wbuf1, rows0, rows1,
                acc, isem0, isem1, gsem0, gsem1, ssem0, ssem1):
    cidx = lax.axis_index("c")
    sid = lax.axis_index("s")
    base = sid * ACC_ROWS
    e0 = sid * EDGES_PER_SUB
    d0 = sid * (EDGES_PER_SUB // MB)
    sbufs = (sbuf0, sbuf1)
    dbufs = (dbuf0, dbuf1)
    wbufs = (wbuf0, wbuf1)
    rowss = (rows0, rows1)
    isems = (isem0, isem1)
    gsems = (gsem0, gsem1)
    ssems = (ssem0, ssem1)
    nt = TILES_PER_SUB

    def in_descs(c, t, b):
        toff = e0 + t * TILE
        return (
            pltpu.make_async_copy(srcs4.at[c].at[pl.ds(toff, TILE)],
                                  sbufs[b], isems[b]),
            pltpu.make_async_copy(dst2.at[pl.ds(d0 + t * NMB, NMB)], dbufs[b],
                                  isems[b]),
            pltpu.make_async_copy(wv.at[pl.ds(toff, TILE)], wbufs[b],
                                  isems[b]),
        )

    def scat_descs(b):
        return [
            pltpu.make_async_copy(rowss[b].at[pl.ds(j * MB, MB)],
                                  acc.at[dbufs[b].at[j]], ssems[b])
            for j in range(NMB)
        ]

    for layer in range(NLAYER):
        srcx = (xf, o1, o2)[layer]
        outx = (o1, o2, o3)[layer]
        for step in range(NCHUNK // NSC):
            c = step * NSC + cidx
            cn = c * N_PAD
            pltpu.sync_copy(zhbm, acc.at[pl.ds(base, ACC_ROWS)])
            plsc.subcore_barrier()

            for d in in_descs(c, 0, 0):
                d.start()

            def pair_body(k, carry):
                for b in (0, 1):
                    t = 2 * k + b
                    for d in in_descs(c, t, b):
                        d.wait()

                    gathers = [
                        pltpu.async_copy(
                            srcx.at[sbufs[b].at[pl.ds(j * MB, MB)]],
                            rowss[b].at[pl.ds(j * MB, MB)], gsems[b])
                        for j in range(NMB)
                    ]

                    @pl.when(t >= 1)
                    def _():
                        for d in scat_descs(1 - b):
                            d.wait()

                    @pl.when(t < nt - 1)
                    def _():
                        for d in in_descs(c, t + 1, 1 - b):
                            d.start()

                    for cp in gathers:
                        cp.wait()

                    rw = rowss[b]
                    wb = wbufs[b]

                    def mul(g, carry2):
                        wv16 = wb[pl.ds(g * 16, 16)]
                        for l in range(16):
                            e = g * 16 + l
                            rw[e, :] = rw[e, :] * wv16[l]
                        return carry2
                    lax.fori_loop(0, TILE // 16, mul, 0)

                    for d in scat_descs(b):
                        d.start(add=True)
                return carry
            lax.fori_loop(0, nt // 2, pair_body, 0)
            for d in scat_descs((nt - 1) % 2):
                d.wait()
            plsc.subcore_barrier()
            pltpu.sync_copy(acc.at[pl.ds(base, ACC_ROWS)],
                            outx.at[pl.ds(cn + base, ACC_ROWS)])
            plsc.subcore_barrier()


_ods = jax.ShapeDtypeStruct((NCHUNK * N_PAD, 16), jnp.float32)
_spmm3_sc = pl.kernel(
    _spmm3_body,
    out_type=(_ods, _ods, _ods),
    mesh=plsc.VectorSubcoreMesh(core_axis_name="c", subcore_axis_name="s"),
    compiler_params=pltpu.CompilerParams(use_tc_tiling_on_sc=False),
    scratch_types=[
        pltpu.VMEM((TILE,), jnp.int32),
        pltpu.VMEM((TILE,), jnp.int32),
        pltpu.VMEM((NMB, MB), jnp.int32),
        pltpu.VMEM((NMB, MB), jnp.int32),
        pltpu.VMEM((TILE,), jnp.float32),
        pltpu.VMEM((TILE,), jnp.float32),
        pltpu.VMEM((TILE, 16), jnp.float32),
        pltpu.VMEM((TILE, 16), jnp.float32),
        pltpu.VMEM_SHARED((N_PAD, 16), jnp.float32),
        pltpu.SemaphoreType.DMA,
        pltpu.SemaphoreType.DMA,
        pltpu.SemaphoreType.DMA,
        pltpu.SemaphoreType.DMA,
        pltpu.SemaphoreType.DMA,
        pltpu.SemaphoreType.DMA,
    ],
)


def _tangent_body(w_ref, o_ref):
    w = w_ref[...]
    y = w[:, 1:]
    y_sq = jnp.sum(y * y, axis=1, keepdims=True)
    x0 = jnp.sqrt(jnp.clip(1.0 + y_sq, EPS, None))
    y_norm = jnp.clip(jnp.sqrt(y_sq), MIN_NORM, None)
    theta = jnp.clip(x0, 1.0 + EPS, None)
    acosh = jnp.log(theta + jnp.sqrt(theta * theta - 1.0))
    rest = acosh * y / y_norm
    x64 = jnp.concatenate(
        [jnp.zeros_like(x0), rest,
         jnp.zeros((rest.shape[0], NCHUNK * 16 - EMB_DIM), rest.dtype)],
        axis=1)
    for cc in range(NCHUNK):
        o_ref[cc, :, :] = x64[:, cc * 16:(cc + 1) * 16]


def _decode_body(h1_ref, h2_ref, h3_ref, o_ref):
    h64 = jnp.concatenate(
        [h1_ref[cc, :, :] + h2_ref[cc, :, :] + h3_ref[cc, :, :]
         for cc in range(NCHUNK)], axis=1)
    x = h64[:, 1:EMB_DIM]
    x_sq = jnp.sum(x * x, axis=1, keepdims=True)
    x_norm = jnp.clip(jnp.sqrt(x_sq), MIN_NORM, None)
    e = jnp.exp(x_norm)
    ei = 1.0 / e
    sinh = 0.5 * (e - ei)
    rest = sinh * x / x_norm
    r_sq = jnp.sum(rest * rest, axis=1, keepdims=True)
    x0 = jnp.sqrt(jnp.clip(1.0 + r_sq, EPS, None))
    o_ref[...] = jnp.concatenate([x0, rest], axis=1)


_tangent_tc = pl.pallas_call(
    _tangent_body,
    grid=(N_PAD // TROWS,),
    in_specs=[pl.BlockSpec((TROWS, EMB_DIM), lambda i: (i, 0))],
    out_specs=pl.BlockSpec((NCHUNK, TROWS, 16), lambda i: (0, i, 0)),
    out_shape=jax.ShapeDtypeStruct((NCHUNK, N_PAD, 16), jnp.float32),
)

_decode_tc = pl.pallas_call(
    _decode_body,
    grid=(N_PAD // TROWS,),
    in_specs=[pl.BlockSpec((NCHUNK, TROWS, 16), lambda i: (0, i, 0))
              for _ in range(NLAYER)],
    out_specs=pl.BlockSpec((TROWS, EMB_DIM), lambda i: (i, 0)),
    out_shape=jax.ShapeDtypeStruct((N_NODES, EMB_DIM), jnp.float32),
)


@jax.jit
def kernel(weight, edge_index, edge_weight):
    pad = E_PAD - N_EDGES
    src = jnp.pad(edge_index[0].astype(jnp.int32), (0, pad))
    srcs4 = src[None, :] + (jnp.arange(NCHUNK, dtype=jnp.int32)
                            * N_PAD)[:, None]
    dst2 = jnp.pad(edge_index[1].astype(jnp.int32), (0, pad)).reshape(
        E_PAD // MB, MB)
    w = jnp.pad(edge_weight, (0, pad))
    zhbm = jnp.zeros((ACC_ROWS, 16), jnp.float32)

    xf = _tangent_tc(weight).reshape(NCHUNK * N_PAD, 16)
    o1, o2, o3 = _spmm3_sc(xf, srcs4, dst2, w, zhbm)
    return _decode_tc(o1.reshape(NCHUNK, N_PAD, 16),
                      o2.reshape(NCHUNK, N_PAD, 16),
                      o3.reshape(NCHUNK, N_PAD, 16))
